# Initial kernel scaffold; baseline (speedup 1.0000x reference)
import jax
import jax.numpy as jnp
from jax.experimental import pallas as pl


def kernel(z, edge_index, edge_attr, W1_0, b1_0, W2_0, b2_0, We1_0, be1_0, We2_0, be2_0, W1_1, b1_1, W2_1, b2_1, We1_1, be1_1, We2_1, be2_1):
    raise NotImplementedError("write your pallas kernel here")


# speedup vs baseline: 3.0039x; 3.0039x over previous
import functools
import jax
import jax.numpy as jnp
from jax import lax
from jax.experimental import pallas as pl
from jax.experimental.pallas import tpu as pltpu
from jax.experimental.pallas import tpu_sc as plsc

_NC = 2
_NS = 16
_C = 80


def _edge_mlp(ea, We1, be1, We2, be2):
    E, F = ea.shape
    D = We1.shape[1]
    BE = 2000
    assert E % BE == 0

    def body(ea_ref, w1_ref, b1_ref, w2_ref, b2_ref, o_ref):
        ea_b = ea_ref[...]
        u = jnp.broadcast_to(b1_ref[...], (BE, D))
        for k in range(F):
            u = u + ea_b[:, k:k + 1] * w1_ref[k:k + 1, :]
        h = jnp.maximum(u, 0.0)
        o_ref[...] = (
            jnp.dot(h, w2_ref[...], preferred_element_type=jnp.float32)
            + b2_ref[...]
        )

    return pl.pallas_call(
        body,
        grid=(E // BE,),
        in_specs=[
            pl.BlockSpec((BE, F), lambda i: (i, 0)),
            pl.BlockSpec((F, D), lambda i: (0, 0)),
            pl.BlockSpec((1, D), lambda i: (0, 0)),
            pl.BlockSpec((D, D), lambda i: (0, 0)),
            pl.BlockSpec((1, D), lambda i: (0, 0)),
        ],
        out_specs=pl.BlockSpec((BE, D), lambda i: (i, 0)),
        out_shape=jax.ShapeDtypeStruct((E, D), jnp.float32),
    )(ea, We1, be1.reshape(1, D), We2, be2.reshape(1, D))


def _sc_message_pass(out_nodes, m_edges, ii3d, jj3d):
    N, D = out_nodes.shape
    NW, CHW, C = ii3d.shape
    EW = CHW * C
    NPAD = ((N + 2048 - 1) // 2048) * 2048
    stripe = NPAD // _NS
    assert stripe % C == 0
    mesh = plsc.VectorSubcoreMesh(core_axis_name="c", subcore_axis_name="s")

    @functools.partial(
        pl.kernel,
        out_type=jax.ShapeDtypeStruct((_NC * NPAD, D), jnp.float32),
        mesh=mesh,
        scratch_types=[
            pltpu.VMEM((CHW, C), jnp.int32),
            pltpu.VMEM((1, C), jnp.int32),
            pltpu.VMEM((C, D), jnp.float32),
            pltpu.VMEM((C, D), jnp.float32),
            pltpu.VMEM_SHARED((NPAD, D), jnp.float32),
            pltpu.SemaphoreType.DMA,
            pltpu.SemaphoreType.DMA,
        ],
    )
    def k(out_hbm, m_hbm, ii_hbm, jj_hbm, part_hbm,
          ii_v, jj_v, g_v, m_v, acc, sem_g, sem_m):
        cid = lax.axis_index("c")
        sid = lax.axis_index("s")
        wid = sid * _NC + cid

        def zrow(r, carry):
            for kk in range(D // 16):
                g_v[r, pl.ds(kk * 16, 16)] = jnp.zeros((16,), jnp.float32)
            return carry
        lax.fori_loop(0, C, zrow, 0)
        for q in range(stripe // C):
            pltpu.sync_copy(g_v, acc.at[pl.ds(sid * stripe + q * C, C)])

        pltpu.sync_copy(ii_hbm.at[wid], ii_v)
        plsc.subcore_barrier()

        def body(t, carry):
            pltpu.sync_copy(jj_hbm.at[wid * CHW + t], jj_v)
            cp_g = pltpu.async_copy(out_hbm.at[jj_v.at[0]], g_v, sem_g)
            cp_m = pltpu.async_copy(
                m_hbm.at[pl.ds(wid * EW + t * C, C)], m_v, sem_m)
            cp_g.wait()
            pltpu.sync_copy(g_v, acc.at[ii_v.at[t]], add=True)
            cp_m.wait()
            pltpu.sync_copy(m_v, acc.at[ii_v.at[t]], add=True)
            return carry
        lax.fori_loop(0, CHW, body, 0)
        plsc.subcore_barrier()

        for q in range(stripe // C):
            base = sid * stripe + q * C
            pltpu.sync_copy(acc.at[pl.ds(base, C)],
                            part_hbm.at[pl.ds(cid * NPAD + base, C)])

    return k(out_nodes, m_edges, ii3d, jj3d)


def _node_mlp(out_nodes, p0, p1, W1a, W1b, b1, W2, b2):
    N, D = out_nodes.shape
    BN = 1000
    assert N % BN == 0

    def body(o_ref, p0_ref, p1_ref, w1a_ref, w1b_ref, b1_ref, w2_ref,
             b2_ref, y_ref):
        x = o_ref[...]
        aggr = p0_ref[...] + p1_ref[...]
        h = jnp.maximum(
            jnp.dot(x, w1a_ref[...], preferred_element_type=jnp.float32)
            + jnp.dot(aggr, w1b_ref[...], preferred_element_type=jnp.float32)
            + b1_ref[...], 0.0)
        y_ref[...] = (
            x + jnp.dot(h, w2_ref[...], preferred_element_type=jnp.float32)
            + b2_ref[...]
        )

    return pl.pallas_call(
        body,
        grid=(N // BN,),
        in_specs=[
            pl.BlockSpec((BN, D), lambda i: (i, 0)),
            pl.BlockSpec((BN, D), lambda i: (i, 0)),
            pl.BlockSpec((BN, D), lambda i: (i, 0)),
            pl.BlockSpec((D, D), lambda i: (0, 0)),
            pl.BlockSpec((D, D), lambda i: (0, 0)),
            pl.BlockSpec((1, D), lambda i: (0, 0)),
            pl.BlockSpec((D, D), lambda i: (0, 0)),
            pl.BlockSpec((1, D), lambda i: (0, 0)),
        ],
        out_specs=pl.BlockSpec((BN, D), lambda i: (i, 0)),
        out_shape=jax.ShapeDtypeStruct((N, D), jnp.float32),
    )(out_nodes, p0, p1, W1a, W1b, b1.reshape(1, D), W2, b2.reshape(1, D))


def kernel(z, edge_index, edge_attr,
           W1_0, b1_0, W2_0, b2_0, We1_0, be1_0, We2_0, be2_0,
           W1_1, b1_1, W2_1, b2_1, We1_1, be1_1, We2_1, be2_1):
    N, D = z.shape
    E = edge_index.shape[1]
    NW = _NC * _NS
    assert E % (NW * _C) == 0
    ii3d = edge_index[0].reshape(NW, E // (NW * _C), _C)
    jj3d = edge_index[1].reshape(E // _C, 1, _C)

    params = [
        (W1_0, b1_0, W2_0, b2_0, We1_0, be1_0, We2_0, be2_0),
        (W1_1, b1_1, W2_1, b2_1, We1_1, be1_1, We2_1, be2_1),
    ]
    msgs = [_edge_mlp(edge_attr, p[4], p[5], p[6], p[7]) for p in params]

    NPAD = ((N + 2048 - 1) // 2048) * 2048
    out = z
    for l, (W1, b1, W2, b2, _, _, _, _) in enumerate(params):
        part = _sc_message_pass(out, msgs[l], ii3d, jj3d)
        out = _node_mlp(out, part[:N], part[NPAD:NPAD + N],
                        W1[:D], W1[D:], b1, W2, b2)
    return out


# speedup vs baseline: 3.9157x; 1.3035x over previous
import functools
import jax
import jax.numpy as jnp
from jax import lax
from jax.experimental import pallas as pl
from jax.experimental.pallas import tpu as pltpu
from jax.experimental.pallas import tpu_sc as plsc

_NC = 2
_NS = 16
_C = 80


def _edge_mlp(ea, We1, be1, We2, be2):
    E, F = ea.shape
    D = We1.shape[1]
    BE = 2000
    assert E % BE == 0

    def body(ea_ref, w1_ref, b1_ref, w2_ref, b2_ref, o_ref):
        ea_b = ea_ref[...]
        u = jnp.broadcast_to(b1_ref[...], (BE, D))
        for k in range(F):
            u = u + ea_b[:, k:k + 1] * w1_ref[k:k + 1, :]
        h = jnp.maximum(u, 0.0)
        o_ref[...] = (
            jnp.dot(h, w2_ref[...], preferred_element_type=jnp.float32)
            + b2_ref[...]
        )

    return pl.pallas_call(
        body,
        grid=(E // BE,),
        in_specs=[
            pl.BlockSpec((BE, F), lambda i: (i, 0)),
            pl.BlockSpec((F, D), lambda i: (0, 0)),
            pl.BlockSpec((1, D), lambda i: (0, 0)),
            pl.BlockSpec((D, D), lambda i: (0, 0)),
            pl.BlockSpec((1, D), lambda i: (0, 0)),
        ],
        out_specs=pl.BlockSpec((BE, D), lambda i: (i, 0)),
        out_shape=jax.ShapeDtypeStruct((E, D), jnp.float32),
    )(ea, We1, be1.reshape(1, D), We2, be2.reshape(1, D))


def _sc_message_pass(out_nodes, m_edges, idx2):
    N, D = out_nodes.shape
    NCH, two, C = idx2.shape
    NW = _NC * _NS
    CHW = NCH // NW
    EW = CHW * C
    NPAD = ((N + 2048 - 1) // 2048) * 2048
    stripe = NPAD // _NS
    assert stripe % C == 0
    mesh = plsc.VectorSubcoreMesh(core_axis_name="c", subcore_axis_name="s")

    @functools.partial(
        pl.kernel,
        out_type=jax.ShapeDtypeStruct((_NC * NPAD, D), jnp.float32),
        mesh=mesh,
        scratch_types=[
            pltpu.VMEM((2, C), jnp.int32),
            pltpu.VMEM((2, C), jnp.int32),
            pltpu.VMEM((C, D), jnp.float32),
            pltpu.VMEM((C, D), jnp.float32),
            pltpu.VMEM((C, D), jnp.float32),
            pltpu.VMEM((C, D), jnp.float32),
            pltpu.VMEM_SHARED((NPAD, D), jnp.float32),
        ] + [pltpu.SemaphoreType.DMA] * 10,
    )
    def k(out_hbm, m_hbm, idx_hbm, part_hbm,
          b0, b1, g0, g1, m0, m1, acc,
          si0, si1, sg0, sg1, sm0, sm1, tg0, tg1, tm0, tm1):
        cid = lax.axis_index("c")
        sid = lax.axis_index("s")
        wid = sid * _NC + cid
        b = [b0, b1]
        g = [g0, g1]
        m = [m0, m1]
        si = [si0, si1]
        sg = [sg0, sg1]
        sm = [sm0, sm1]
        tg = [tg0, tg1]
        tm = [tm0, tm1]

        def zrow(r, carry):
            for kk in range(D // 16):
                g0[r, pl.ds(kk * 16, 16)] = jnp.zeros((16,), jnp.float32)
            return carry
        lax.fori_loop(0, C, zrow, 0)
        for q in range(stripe // C):
            pltpu.sync_copy(g0, acc.at[pl.ds(sid * stripe + q * C, C)])
        plsc.subcore_barrier()

        def stage_idx(t, sl):
            pltpu.async_copy(idx_hbm.at[wid * CHW + t], b[sl], si[sl])

        def wait_idx(sl):
            pltpu.make_async_copy(idx_hbm.at[0], b[sl], si[sl]).wait()

        def stage_data(t, sl):
            pltpu.async_copy(out_hbm.at[b[sl].at[1]], g[sl], sg[sl])
            pltpu.async_copy(m_hbm.at[pl.ds(wid * EW + t * C, C)],
                             m[sl], sm[sl])

        def wait_data(sl):
            pltpu.make_async_copy(m_hbm.at[pl.ds(0, C)], g[sl],
                                  sg[sl]).wait()
            pltpu.make_async_copy(m_hbm.at[pl.ds(0, C)], m[sl],
                                  sm[sl]).wait()

        def scat(sl):
            pltpu.async_copy(g[sl], acc.at[b[sl].at[0]], tg[sl], add=True)
            pltpu.async_copy(m[sl], acc.at[b[sl].at[0]], tm[sl], add=True)

        def wait_scat(sl):
            pltpu.make_async_copy(m_hbm.at[pl.ds(0, C)], g[sl],
                                  tg[sl]).wait()
            pltpu.make_async_copy(m_hbm.at[pl.ds(0, C)], m[sl],
                                  tm[sl]).wait()

        stage_idx(0, 0)
        stage_idx(1, 1)
        wait_idx(0)
        stage_data(0, 0)
        wait_idx(1)
        stage_data(1, 1)

        def body(k2, carry):
            t0 = 2 * k2
            wait_data(0)
            scat(0)
            wait_scat(0)
            stage_idx(t0 + 2, 0)
            wait_idx(0)
            stage_data(t0 + 2, 0)
            wait_data(1)
            scat(1)
            wait_scat(1)
            stage_idx(t0 + 3, 1)
            wait_idx(1)
            stage_data(t0 + 3, 1)
            return carry
        assert CHW % 2 == 1
        lax.fori_loop(0, CHW // 2 - 1, body, 0)

        wait_data(0)
        scat(0)
        wait_scat(0)
        stage_idx(CHW - 1, 0)
        wait_idx(0)
        stage_data(CHW - 1, 0)
        wait_data(1)
        scat(1)
        wait_scat(1)
        wait_data(0)
        scat(0)
        wait_scat(0)
        plsc.subcore_barrier()

        for q in range(stripe // C):
            base = sid * stripe + q * C
            pltpu.sync_copy(acc.at[pl.ds(base, C)],
                            part_hbm.at[pl.ds(cid * NPAD + base, C)])

    return k(out_nodes, m_edges, idx2)


def _node_mlp(out_nodes, p0, p1, W1a, W1b, b1, W2, b2):
    N, D = out_nodes.shape
    BN = 1000
    assert N % BN == 0

    def body(o_ref, p0_ref, p1_ref, w1a_ref, w1b_ref, b1_ref, w2_ref,
             b2_ref, y_ref):
        x = o_ref[...]
        aggr = p0_ref[...] + p1_ref[...]
        h = jnp.maximum(
            jnp.dot(x, w1a_ref[...], preferred_element_type=jnp.float32)
            + jnp.dot(aggr, w1b_ref[...], preferred_element_type=jnp.float32)
            + b1_ref[...], 0.0)
        y_ref[...] = (
            x + jnp.dot(h, w2_ref[...], preferred_element_type=jnp.float32)
            + b2_ref[...]
        )

    return pl.pallas_call(
        body,
        grid=(N // BN,),
        in_specs=[
            pl.BlockSpec((BN, D), lambda i: (i, 0)),
            pl.BlockSpec((BN, D), lambda i: (i, 0)),
            pl.BlockSpec((BN, D), lambda i: (i, 0)),
            pl.BlockSpec((D, D), lambda i: (0, 0)),
            pl.BlockSpec((D, D), lambda i: (0, 0)),
            pl.BlockSpec((1, D), lambda i: (0, 0)),
            pl.BlockSpec((D, D), lambda i: (0, 0)),
            pl.BlockSpec((1, D), lambda i: (0, 0)),
        ],
        out_specs=pl.BlockSpec((BN, D), lambda i: (i, 0)),
        out_shape=jax.ShapeDtypeStruct((N, D), jnp.float32),
    )(out_nodes, p0, p1, W1a, W1b, b1.reshape(1, D), W2, b2.reshape(1, D))


def kernel(z, edge_index, edge_attr,
           W1_0, b1_0, W2_0, b2_0, We1_0, be1_0, We2_0, be2_0,
           W1_1, b1_1, W2_1, b2_1, We1_1, be1_1, We2_1, be2_1):
    N, D = z.shape
    E = edge_index.shape[1]
    NW = _NC * _NS
    assert E % (NW * _C) == 0
    idx2 = jnp.transpose(edge_index.reshape(2, E // _C, _C), (1, 0, 2))

    params = [
        (W1_0, b1_0, W2_0, b2_0, We1_0, be1_0, We2_0, be2_0),
        (W1_1, b1_1, W2_1, b2_1, We1_1, be1_1, We2_1, be2_1),
    ]
    msgs = [_edge_mlp(edge_attr, p[4], p[5], p[6], p[7]) for p in params]

    NPAD = ((N + 2048 - 1) // 2048) * 2048
    out = z
    for l, (W1, b1, W2, b2, _, _, _, _) in enumerate(params):
        part = _sc_message_pass(out, msgs[l], idx2)
        out = _node_mlp(out, part[:N], part[NPAD:NPAD + N],
                        W1[:D], W1[D:], b1, W2, b2)
    return out


# speedup vs baseline: 3.9868x; 1.0182x over previous
import functools
import jax
import jax.numpy as jnp
from jax import lax
from jax.experimental import pallas as pl
from jax.experimental.pallas import tpu as pltpu
from jax.experimental.pallas import tpu_sc as plsc

_NC = 2
_NS = 16
_C = 80


def _edge_mlps(ea, p0, p1):
    E, F = ea.shape
    D = p0[0].shape[1]
    BE = 2000
    assert E % BE == 0

    def body(ea_ref, w1a_ref, b1a_ref, w2a_ref, b2a_ref,
             w1b_ref, b1b_ref, w2b_ref, b2b_ref, oa_ref, ob_ref):
        ea_b = ea_ref[...]

        def one(w1_ref, b1_ref, w2_ref, b2_ref, o_ref):
            u = jnp.broadcast_to(b1_ref[...], (BE, D))
            for k in range(F):
                u = u + ea_b[:, k:k + 1] * w1_ref[k:k + 1, :]
            h = jnp.maximum(u, 0.0)
            o_ref[...] = (
                jnp.dot(h, w2_ref[...], preferred_element_type=jnp.float32)
                + b2_ref[...]
            )
        one(w1a_ref, b1a_ref, w2a_ref, b2a_ref, oa_ref)
        one(w1b_ref, b1b_ref, w2b_ref, b2b_ref, ob_ref)

    wspec = [
        pl.BlockSpec((F, D), lambda i: (0, 0)),
        pl.BlockSpec((1, D), lambda i: (0, 0)),
        pl.BlockSpec((D, D), lambda i: (0, 0)),
        pl.BlockSpec((1, D), lambda i: (0, 0)),
    ]
    return pl.pallas_call(
        body,
        grid=(E // BE,),
        in_specs=[pl.BlockSpec((BE, F), lambda i: (i, 0))] + wspec + wspec,
        out_specs=[pl.BlockSpec((BE, D), lambda i: (i, 0))] * 2,
        out_shape=[jax.ShapeDtypeStruct((E, D), jnp.float32)] * 2,
    )(ea,
      p0[0], p0[1].reshape(1, D), p0[2], p0[3].reshape(1, D),
      p1[0], p1[1].reshape(1, D), p1[2], p1[3].reshape(1, D))


def _sc_message_pass(out_nodes, m_edges, idx2):
    N, D = out_nodes.shape
    NCH, two, C = idx2.shape
    NW = _NC * _NS
    CHW = NCH // NW
    EW = CHW * C
    NPAD = ((N + 2048 - 1) // 2048) * 2048
    stripe = NPAD // _NS
    assert stripe % C == 0
    mesh = plsc.VectorSubcoreMesh(core_axis_name="c", subcore_axis_name="s")

    @functools.partial(
        pl.kernel,
        out_type=jax.ShapeDtypeStruct((_NC * NPAD, D), jnp.float32),
        mesh=mesh,
        scratch_types=[
            pltpu.VMEM((2, C), jnp.int32),
            pltpu.VMEM((2, C), jnp.int32),
            pltpu.VMEM((C, D), jnp.float32),
            pltpu.VMEM((C, D), jnp.float32),
            pltpu.VMEM((C, D), jnp.float32),
            pltpu.VMEM((C, D), jnp.float32),
            pltpu.VMEM_SHARED((NPAD, D), jnp.float32),
        ] + [pltpu.SemaphoreType.DMA] * 10,
    )
    def k(out_hbm, m_hbm, idx_hbm, part_hbm,
          b0, b1, g0, g1, m0, m1, acc,
          si0, si1, sg0, sg1, sm0, sm1, tg0, tg1, tm0, tm1):
        cid = lax.axis_index("c")
        sid = lax.axis_index("s")
        wid = sid * _NC + cid
        b = [b0, b1]
        g = [g0, g1]
        m = [m0, m1]
        si = [si0, si1]
        sg = [sg0, sg1]
        sm = [sm0, sm1]
        tg = [tg0, tg1]
        tm = [tm0, tm1]

        def zrow(r, carry):
            for kk in range(D // 16):
                g0[r, pl.ds(kk * 16, 16)] = jnp.zeros((16,), jnp.float32)
            return carry
        lax.fori_loop(0, C, zrow, 0)
        for q in range(stripe // C):
            pltpu.sync_copy(g0, acc.at[pl.ds(sid * stripe + q * C, C)])
        plsc.subcore_barrier()

        def stage_idx(t, sl):
            pltpu.async_copy(idx_hbm.at[wid * CHW + t], b[sl], si[sl])

        def wait_idx(sl):
            pltpu.make_async_copy(idx_hbm.at[0], b[sl], si[sl]).wait()

        def stage_data(t, sl):
            pltpu.async_copy(out_hbm.at[b[sl].at[1]], g[sl], sg[sl])
            pltpu.async_copy(m_hbm.at[pl.ds(wid * EW + t * C, C)],
                             m[sl], sm[sl])

        def wait_data(sl):
            pltpu.make_async_copy(m_hbm.at[pl.ds(0, C)], g[sl],
                                  sg[sl]).wait()
            pltpu.make_async_copy(m_hbm.at[pl.ds(0, C)], m[sl],
                                  sm[sl]).wait()

        def scat(sl):
            pltpu.async_copy(g[sl], acc.at[b[sl].at[0]], tg[sl], add=True)
            pltpu.async_copy(m[sl], acc.at[b[sl].at[0]], tm[sl], add=True)

        def wait_scat(sl):
            pltpu.make_async_copy(m_hbm.at[pl.ds(0, C)], g[sl],
                                  tg[sl]).wait()
            pltpu.make_async_copy(m_hbm.at[pl.ds(0, C)], m[sl],
                                  tm[sl]).wait()

        stage_idx(0, 0)
        stage_idx(1, 1)
        wait_idx(0)
        stage_data(0, 0)
        wait_idx(1)
        stage_data(1, 1)

        def body(k2, carry):
            t0 = 2 * k2
            wait_data(0)
            scat(0)
            wait_scat(0)
            stage_idx(t0 + 2, 0)
            wait_idx(0)
            stage_data(t0 + 2, 0)
            wait_data(1)
            scat(1)
            wait_scat(1)
            stage_idx(t0 + 3, 1)
            wait_idx(1)
            stage_data(t0 + 3, 1)
            return carry
        assert CHW % 2 == 1
        lax.fori_loop(0, CHW // 2 - 1, body, 0)

        wait_data(0)
        scat(0)
        wait_scat(0)
        stage_idx(CHW - 1, 0)
        wait_idx(0)
        stage_data(CHW - 1, 0)
        wait_data(1)
        scat(1)
        wait_scat(1)
        wait_data(0)
        scat(0)
        wait_scat(0)
        plsc.subcore_barrier()

        for q in range(stripe // C):
            base = sid * stripe + q * C
            pltpu.sync_copy(acc.at[pl.ds(base, C)],
                            part_hbm.at[pl.ds(cid * NPAD + base, C)])

    return k(out_nodes, m_edges, idx2)


def _node_mlp(out_nodes, p0, p1, W1a, W1b, b1, W2, b2):
    N, D = out_nodes.shape
    BN = 1000
    assert N % BN == 0

    def body(o_ref, p0_ref, p1_ref, w1a_ref, w1b_ref, b1_ref, w2_ref,
             b2_ref, y_ref):
        x = o_ref[...]
        aggr = p0_ref[...] + p1_ref[...]
        h = jnp.maximum(
            jnp.dot(x, w1a_ref[...], preferred_element_type=jnp.float32)
            + jnp.dot(aggr, w1b_ref[...], preferred_element_type=jnp.float32)
            + b1_ref[...], 0.0)
        y_ref[...] = (
            x + jnp.dot(h, w2_ref[...], preferred_element_type=jnp.float32)
            + b2_ref[...]
        )

    return pl.pallas_call(
        body,
        grid=(N // BN,),
        in_specs=[
            pl.BlockSpec((BN, D), lambda i: (i, 0)),
            pl.BlockSpec((BN, D), lambda i: (i, 0)),
            pl.BlockSpec((BN, D), lambda i: (i, 0)),
            pl.BlockSpec((D, D), lambda i: (0, 0)),
            pl.BlockSpec((D, D), lambda i: (0, 0)),
            pl.BlockSpec((1, D), lambda i: (0, 0)),
            pl.BlockSpec((D, D), lambda i: (0, 0)),
            pl.BlockSpec((1, D), lambda i: (0, 0)),
        ],
        out_specs=pl.BlockSpec((BN, D), lambda i: (i, 0)),
        out_shape=jax.ShapeDtypeStruct((N, D), jnp.float32),
    )(out_nodes, p0, p1, W1a, W1b, b1.reshape(1, D), W2, b2.reshape(1, D))


def kernel(z, edge_index, edge_attr,
           W1_0, b1_0, W2_0, b2_0, We1_0, be1_0, We2_0, be2_0,
           W1_1, b1_1, W2_1, b2_1, We1_1, be1_1, We2_1, be2_1):
    N, D = z.shape
    E = edge_index.shape[1]
    NW = _NC * _NS
    assert E % (NW * _C) == 0
    idx2 = jnp.transpose(edge_index.reshape(2, E // _C, _C), (1, 0, 2))

    params = [
        (W1_0, b1_0, W2_0, b2_0, We1_0, be1_0, We2_0, be2_0),
        (W1_1, b1_1, W2_1, b2_1, We1_1, be1_1, We2_1, be2_1),
    ]
    msgs = _edge_mlps(edge_attr,
                      (We1_0, be1_0, We2_0, be2_0),
                      (We1_1, be1_1, We2_1, be2_1))

    NPAD = ((N + 2048 - 1) // 2048) * 2048
    out = z
    for l, (W1, b1, W2, b2, _, _, _, _) in enumerate(params):
        part = _sc_message_pass(out, msgs[l], idx2)
        out = _node_mlp(out, part[:N], part[NPAD:NPAD + N],
                        W1[:D], W1[D:], b1, W2, b2)
    return out


# speedup vs baseline: 4.0385x; 1.0130x over previous
import functools
import jax
import jax.numpy as jnp
from jax import lax
from jax.experimental import pallas as pl
from jax.experimental.pallas import tpu as pltpu
from jax.experimental.pallas import tpu_sc as plsc

_NC = 2
_NS = 16
_C = 80


def _edge_mlps(ea, p0, p1):
    E, F = ea.shape
    D = p0[0].shape[1]
    BE = 2000
    assert E % BE == 0

    def body(ea_ref, w1a_ref, b1a_ref, w2a_ref, b2a_ref,
             w1b_ref, b1b_ref, w2b_ref, b2b_ref, oa_ref, ob_ref):
        ea_b = ea_ref[...]

        def one(w1_ref, b1_ref, w2_ref, b2_ref, o_ref):
            u = jnp.broadcast_to(b1_ref[...], (BE, D))
            for k in range(F):
                u = u + ea_b[:, k:k + 1] * w1_ref[k:k + 1, :]
            h = jnp.maximum(u, 0.0)
            o_ref[...] = (
                jnp.dot(h, w2_ref[...], preferred_element_type=jnp.float32)
                + b2_ref[...]
            )
        one(w1a_ref, b1a_ref, w2a_ref, b2a_ref, oa_ref)
        one(w1b_ref, b1b_ref, w2b_ref, b2b_ref, ob_ref)

    wspec = [
        pl.BlockSpec((F, D), lambda i: (0, 0)),
        pl.BlockSpec((1, D), lambda i: (0, 0)),
        pl.BlockSpec((D, D), lambda i: (0, 0)),
        pl.BlockSpec((1, D), lambda i: (0, 0)),
    ]
    return pl.pallas_call(
        body,
        grid=(E // BE,),
        in_specs=[pl.BlockSpec((BE, F), lambda i: (i, 0))] + wspec + wspec,
        out_specs=[pl.BlockSpec((BE, D), lambda i: (i, 0))] * 2,
        out_shape=[jax.ShapeDtypeStruct((E, D), jnp.float32)] * 2,
    )(ea,
      p0[0], p0[1].reshape(1, D), p0[2], p0[3].reshape(1, D),
      p1[0], p1[1].reshape(1, D), p1[2], p1[3].reshape(1, D))


def _sc_message_pass(out_nodes, m_edges, idx2):
    N, D = out_nodes.shape
    NCH, two, C = idx2.shape
    NW = _NC * _NS
    CHW = NCH // NW
    EW = CHW * C
    NPAD = ((N + 2048 - 1) // 2048) * 2048
    stripe = NPAD // _NS
    assert stripe % C == 0
    mesh = plsc.VectorSubcoreMesh(core_axis_name="c", subcore_axis_name="s")

    @functools.partial(
        pl.kernel,
        out_type=jax.ShapeDtypeStruct((_NC * NPAD, D), jnp.float32),
        mesh=mesh,
        scratch_types=[
            pltpu.VMEM((2, C), jnp.int32),
            pltpu.VMEM((2, C), jnp.int32),
            pltpu.VMEM((2, C), jnp.int32),
            pltpu.VMEM((2, C), jnp.int32),
            pltpu.VMEM((C, D), jnp.float32),
            pltpu.VMEM((C, D), jnp.float32),
            pltpu.VMEM((C, D), jnp.float32),
            pltpu.VMEM((C, D), jnp.float32),
            pltpu.VMEM_SHARED((NPAD, D), jnp.float32),
        ] + [pltpu.SemaphoreType.DMA] * 12,
    )
    def k(out_hbm, m_hbm, idx_hbm, part_hbm,
          b0, b1, b2, b3, g0, g1, m0, m1, acc,
          si0, si1, si2, si3, sg0, sg1, sm0, sm1, tg0, tg1, tm0, tm1):
        cid = lax.axis_index("c")
        sid = lax.axis_index("s")
        wid = sid * _NC + cid
        b = [b0, b1, b2, b3]
        g = [g0, g1]
        m = [m0, m1]
        si = [si0, si1, si2, si3]
        sg = [sg0, sg1]
        sm = [sm0, sm1]
        tg = [tg0, tg1]
        tm = [tm0, tm1]

        def zrow(r, carry):
            for kk in range(D // 16):
                g0[r, pl.ds(kk * 16, 16)] = jnp.zeros((16,), jnp.float32)
            return carry
        lax.fori_loop(0, C, zrow, 0)
        for q in range(stripe // C):
            pltpu.sync_copy(g0, acc.at[pl.ds(sid * stripe + q * C, C)])
        plsc.subcore_barrier()

        def stage_idx(t, ir):
            pltpu.async_copy(idx_hbm.at[wid * CHW + t], b[ir], si[ir])

        def wait_idx(ir):
            pltpu.make_async_copy(idx_hbm.at[0], b[ir], si[ir]).wait()

        def stage_data(t, sl, ir):
            pltpu.async_copy(out_hbm.at[b[ir].at[1]], g[sl], sg[sl])
            pltpu.async_copy(m_hbm.at[pl.ds(wid * EW + t * C, C)],
                             m[sl], sm[sl])

        def wait_data(sl):
            pltpu.make_async_copy(m_hbm.at[pl.ds(0, C)], g[sl],
                                  sg[sl]).wait()
            pltpu.make_async_copy(m_hbm.at[pl.ds(0, C)], m[sl],
                                  sm[sl]).wait()

        def scat(sl, ir):
            pltpu.async_copy(g[sl], acc.at[b[ir].at[0]], tg[sl], add=True)
            pltpu.async_copy(m[sl], acc.at[b[ir].at[0]], tm[sl], add=True)

        def wait_scat(sl):
            pltpu.make_async_copy(m_hbm.at[pl.ds(0, C)], g[sl],
                                  tg[sl]).wait()
            pltpu.make_async_copy(m_hbm.at[pl.ds(0, C)], m[sl],
                                  tm[sl]).wait()

        for r in range(4):
            stage_idx(r, r)
        wait_idx(0)
        stage_data(0, 0, 0)
        wait_idx(1)
        stage_data(1, 1, 1)

        def proc(t, sl, ir, nxt_idx, nxt_data):
            wait_data(sl)
            scat(sl, ir)
            wait_scat(sl)
            if nxt_idx:
                stage_idx(t + 4, ir)
            if nxt_data:
                wait_idx((ir + 2) % 4)
                stage_data(t + 2, sl, (ir + 2) % 4)

        def body(k4, carry):
            t0 = 4 * k4
            for u in range(4):
                proc(t0 + u, u % 2, u, True, True)
            return carry
        K = (CHW - 4) // 4
        lax.fori_loop(0, K, body, 0)

        for t in range(4 * K, CHW):
            proc(t, t % 2, t % 4, t + 4 < CHW, t + 2 < CHW)
        plsc.subcore_barrier()

        for q in range(stripe // C):
            base = sid * stripe + q * C
            pltpu.sync_copy(acc.at[pl.ds(base, C)],
                            part_hbm.at[pl.ds(cid * NPAD + base, C)])

    return k(out_nodes, m_edges, idx2)


def _node_mlp(out_nodes, p0, p1, W1a, W1b, b1, W2, b2):
    N, D = out_nodes.shape
    BN = 1000
    assert N % BN == 0

    def body(o_ref, p0_ref, p1_ref, w1a_ref, w1b_ref, b1_ref, w2_ref,
             b2_ref, y_ref):
        x = o_ref[...]
        aggr = p0_ref[...] + p1_ref[...]
        h = jnp.maximum(
            jnp.dot(x, w1a_ref[...], preferred_element_type=jnp.float32)
            + jnp.dot(aggr, w1b_ref[...], preferred_element_type=jnp.float32)
            + b1_ref[...], 0.0)
        y_ref[...] = (
            x + jnp.dot(h, w2_ref[...], preferred_element_type=jnp.float32)
            + b2_ref[...]
        )

    return pl.pallas_call(
        body,
        grid=(N // BN,),
        in_specs=[
            pl.BlockSpec((BN, D), lambda i: (i, 0)),
            pl.BlockSpec((BN, D), lambda i: (i, 0)),
            pl.BlockSpec((BN, D), lambda i: (i, 0)),
            pl.BlockSpec((D, D), lambda i: (0, 0)),
            pl.BlockSpec((D, D), lambda i: (0, 0)),
            pl.BlockSpec((1, D), lambda i: (0, 0)),
            pl.BlockSpec((D, D), lambda i: (0, 0)),
            pl.BlockSpec((1, D), lambda i: (0, 0)),
        ],
        out_specs=pl.BlockSpec((BN, D), lambda i: (i, 0)),
        out_shape=jax.ShapeDtypeStruct((N, D), jnp.float32),
    )(out_nodes, p0, p1, W1a, W1b, b1.reshape(1, D), W2, b2.reshape(1, D))


def kernel(z, edge_index, edge_attr,
           W1_0, b1_0, W2_0, b2_0, We1_0, be1_0, We2_0, be2_0,
           W1_1, b1_1, W2_1, b2_1, We1_1, be1_1, We2_1, be2_1):
    N, D = z.shape
    E = edge_index.shape[1]
    NW = _NC * _NS
    assert E % (NW * _C) == 0
    idx2 = jnp.transpose(edge_index.reshape(2, E // _C, _C), (1, 0, 2))

    params = [
        (W1_0, b1_0, W2_0, b2_0, We1_0, be1_0, We2_0, be2_0),
        (W1_1, b1_1, W2_1, b2_1, We1_1, be1_1, We2_1, be2_1),
    ]
    msgs = _edge_mlps(edge_attr,
                      (We1_0, be1_0, We2_0, be2_0),
                      (We1_1, be1_1, We2_1, be2_1))

    NPAD = ((N + 2048 - 1) // 2048) * 2048
    out = z
    for l, (W1, b1, W2, b2, _, _, _, _) in enumerate(params):
        part = _sc_message_pass(out, msgs[l], idx2)
        out = _node_mlp(out, part[:N], part[NPAD:NPAD + N],
                        W1[:D], W1[D:], b1, W2, b2)
    return out


# speedup vs baseline: 4.1184x; 1.0198x over previous
import functools
import jax
import jax.numpy as jnp
from jax import lax
from jax.experimental import pallas as pl
from jax.experimental.pallas import tpu as pltpu
from jax.experimental.pallas import tpu_sc as plsc

_NC = 2
_NS = 16
_C = 80


def _edge_mlps(ea, p0, p1):
    E, F = ea.shape
    D = p0[0].shape[1]
    BE = 2000
    assert E % BE == 0

    def body(ea_ref, w1a_ref, b1a_ref, w2a_ref, b2a_ref,
             w1b_ref, b1b_ref, w2b_ref, b2b_ref, oa_ref, ob_ref):
        ea_b = ea_ref[...]

        def one(w1_ref, b1_ref, w2_ref, b2_ref, o_ref):
            u = jnp.broadcast_to(b1_ref[...], (BE, D))
            for k in range(F):
                u = u + ea_b[:, k:k + 1] * w1_ref[k:k + 1, :]
            h = jnp.maximum(u, 0.0)
            o_ref[...] = (
                jnp.dot(h, w2_ref[...], preferred_element_type=jnp.float32)
                + b2_ref[...]
            )
        one(w1a_ref, b1a_ref, w2a_ref, b2a_ref, oa_ref)
        one(w1b_ref, b1b_ref, w2b_ref, b2b_ref, ob_ref)

    wspec = [
        pl.BlockSpec((F, D), lambda i: (0, 0)),
        pl.BlockSpec((1, D), lambda i: (0, 0)),
        pl.BlockSpec((D, D), lambda i: (0, 0)),
        pl.BlockSpec((1, D), lambda i: (0, 0)),
    ]
    return pl.pallas_call(
        body,
        grid=(E // BE,),
        in_specs=[pl.BlockSpec((BE, F), lambda i: (i, 0))] + wspec + wspec,
        out_specs=[pl.BlockSpec((BE, D), lambda i: (i, 0))] * 2,
        out_shape=[jax.ShapeDtypeStruct((E, D), jnp.float32)] * 2,
    )(ea,
      p0[0], p0[1].reshape(1, D), p0[2], p0[3].reshape(1, D),
      p1[0], p1[1].reshape(1, D), p1[2], p1[3].reshape(1, D))


def _sc_message_pass(out_nodes, m_edges, idx2):
    N, D = out_nodes.shape
    NCH, two, C = idx2.shape
    NW = _NC * _NS
    CHW = NCH // NW
    EW = CHW * C
    NPAD = ((N + 2048 - 1) // 2048) * 2048
    stripe = NPAD // _NS
    assert stripe % C == 0
    mesh = plsc.VectorSubcoreMesh(core_axis_name="c", subcore_axis_name="s")

    @functools.partial(
        pl.kernel,
        out_type=jax.ShapeDtypeStruct((_NC * NPAD, D), jnp.float32),
        mesh=mesh,
        scratch_types=[
            pltpu.VMEM((2, C), jnp.int32),
            pltpu.VMEM((2, C), jnp.int32),
            pltpu.VMEM((2, C), jnp.int32),
            pltpu.VMEM((2, C), jnp.int32),
            pltpu.VMEM((C, D), jnp.float32),
            pltpu.VMEM((C, D), jnp.float32),
            pltpu.VMEM((C, D), jnp.float32),
            pltpu.VMEM((C, D), jnp.float32),
            pltpu.VMEM_SHARED((NPAD, D), jnp.float32),
        ] + [pltpu.SemaphoreType.DMA] * 12,
    )
    def k(out_hbm, m_hbm, idx_hbm, part_hbm,
          b0, b1, b2, b3, g0, g1, m0, m1, acc,
          si0, si1, si2, si3, sg0, sg1, sm0, sm1, tg0, tg1, tm0, tm1):
        cid = lax.axis_index("c")
        sid = lax.axis_index("s")
        wid = sid * _NC + cid
        b = [b0, b1, b2, b3]
        g = [g0, g1]
        m = [m0, m1]
        si = [si0, si1, si2, si3]
        sg = [sg0, sg1]
        sm = [sm0, sm1]
        tg = [tg0, tg1]
        tm = [tm0, tm1]

        def zrow(r, carry):
            for kk in range(D // 16):
                g0[r, pl.ds(kk * 16, 16)] = jnp.zeros((16,), jnp.float32)
            return carry
        lax.fori_loop(0, C, zrow, 0)
        for q in range(stripe // C):
            pltpu.sync_copy(g0, acc.at[pl.ds(sid * stripe + q * C, C)])
        plsc.subcore_barrier()

        def stage_idx(t, ir):
            pltpu.async_copy(idx_hbm.at[wid * CHW + t], b[ir], si[ir])

        def wait_idx(ir):
            pltpu.make_async_copy(idx_hbm.at[0], b[ir], si[ir]).wait()

        def stage_data(t, sl, ir):
            pltpu.async_copy(out_hbm.at[b[ir].at[1]], g[sl], sg[sl])
            pltpu.async_copy(m_hbm.at[pl.ds(wid * EW + t * C, C)],
                             m[sl], sm[sl])

        def wait_data(sl):
            pltpu.make_async_copy(m_hbm.at[pl.ds(0, C)], g[sl],
                                  sg[sl]).wait()
            pltpu.make_async_copy(m_hbm.at[pl.ds(0, C)], m[sl],
                                  sm[sl]).wait()

        def scat(sl, ir):
            pltpu.async_copy(g[sl], acc.at[b[ir].at[0]], tg[sl], add=True)
            pltpu.async_copy(m[sl], acc.at[b[ir].at[0]], tm[sl], add=True)

        def wait_scat(sl):
            pltpu.make_async_copy(m_hbm.at[pl.ds(0, C)], g[sl],
                                  tg[sl]).wait()
            pltpu.make_async_copy(m_hbm.at[pl.ds(0, C)], m[sl],
                                  tm[sl]).wait()

        for r in range(4):
            stage_idx(r, r)
        wait_idx(0)
        stage_data(0, 0, 0)
        wait_idx(1)
        stage_data(1, 1, 1)

        def proc(t, sl, ir, nxt_idx, nxt_data):
            wait_data(sl)
            scat(sl, ir)
            wait_scat(sl)
            if nxt_idx:
                stage_idx(t + 4, ir)
            if nxt_data:
                wait_idx((ir + 2) % 4)
                stage_data(t + 2, sl, (ir + 2) % 4)

        def body(k4, carry):
            t0 = 4 * k4
            for u in range(4):
                proc(t0 + u, u % 2, u, True, True)
            return carry
        K = (CHW - 4) // 4
        lax.fori_loop(0, K, body, 0)

        for t in range(4 * K, CHW):
            proc(t, t % 2, t % 4, t + 4 < CHW, t + 2 < CHW)
        plsc.subcore_barrier()

        for q in range(stripe // C):
            base = sid * stripe + q * C
            pltpu.sync_copy(acc.at[pl.ds(base, C)],
                            part_hbm.at[pl.ds(cid * NPAD + base, C)])

    return k(out_nodes, m_edges, idx2)


def _node_mlp(out_nodes, part, NPAD, W1a, W1b, b1, W2, b2):
    N, D = out_nodes.shape
    BN = 1024
    assert NPAD % BN == 0
    nb = NPAD // BN
    grid = (N + BN - 1) // BN

    def body(o_ref, p0_ref, p1_ref, w1a_ref, w1b_ref, b1_ref, w2_ref,
             b2_ref, y_ref):
        x = o_ref[...]
        aggr = p0_ref[...] + p1_ref[...]
        h = jnp.maximum(
            jnp.dot(x, w1a_ref[...], preferred_element_type=jnp.float32)
            + jnp.dot(aggr, w1b_ref[...], preferred_element_type=jnp.float32)
            + b1_ref[...], 0.0)
        y_ref[...] = (
            x + jnp.dot(h, w2_ref[...], preferred_element_type=jnp.float32)
            + b2_ref[...]
        )

    return pl.pallas_call(
        body,
        grid=(grid,),
        in_specs=[
            pl.BlockSpec((BN, D), lambda i: (i, 0)),
            pl.BlockSpec((BN, D), lambda i: (i, 0)),
            pl.BlockSpec((BN, D), lambda i: (nb + i, 0)),
            pl.BlockSpec((D, D), lambda i: (0, 0)),
            pl.BlockSpec((D, D), lambda i: (0, 0)),
            pl.BlockSpec((1, D), lambda i: (0, 0)),
            pl.BlockSpec((D, D), lambda i: (0, 0)),
            pl.BlockSpec((1, D), lambda i: (0, 0)),
        ],
        out_specs=pl.BlockSpec((BN, D), lambda i: (i, 0)),
        out_shape=jax.ShapeDtypeStruct((N, D), jnp.float32),
    )(out_nodes, part, part, W1a, W1b, b1.reshape(1, D), W2,
      b2.reshape(1, D))


def kernel(z, edge_index, edge_attr,
           W1_0, b1_0, W2_0, b2_0, We1_0, be1_0, We2_0, be2_0,
           W1_1, b1_1, W2_1, b2_1, We1_1, be1_1, We2_1, be2_1):
    N, D = z.shape
    E = edge_index.shape[1]
    NW = _NC * _NS
    assert E % (NW * _C) == 0
    idx2 = jnp.transpose(edge_index.reshape(2, E // _C, _C), (1, 0, 2))

    params = [
        (W1_0, b1_0, W2_0, b2_0, We1_0, be1_0, We2_0, be2_0),
        (W1_1, b1_1, W2_1, b2_1, We1_1, be1_1, We2_1, be2_1),
    ]
    msgs = _edge_mlps(edge_attr,
                      (We1_0, be1_0, We2_0, be2_0),
                      (We1_1, be1_1, We2_1, be2_1))

    NPAD = ((N + 2048 - 1) // 2048) * 2048
    out = z
    for l, (W1, b1, W2, b2, _, _, _, _) in enumerate(params):
        part = _sc_message_pass(out, msgs[l], idx2)
        out = _node_mlp(out, part, NPAD, W1[:D], W1[D:], b1, W2, b2)
    return out
